# baseline (speedup 1.0000x reference)
import functools

import jax
import jax.numpy as jnp
from jax import lax
from jax.experimental import pallas as pl
from jax.experimental.pallas import tpu as pltpu
from jax.experimental.pallas import tpu_sc as plsc

BATCH = 16384
EMBED_DIM = 32
LANES = 16
NROWS = 1000000

_info = plsc.get_sparse_core_info()
NC, NS = _info.num_cores, _info.num_subcores
NW = NC * NS
BPW = BATCH // NW
GROUPS = BPW // LANES


def _mf_body(uids, iids, eut, eit, bu, bi, gb, out,
             uid_v, iid_v, uidx, iidx, ucols, icols, bu_v, bi_v, gb_v,
             out_v, sem_u, sem_i, sem_bu, sem_bi):
    wid = lax.axis_index("s") * NC + lax.axis_index("c")
    base = wid * BPW

    pltpu.sync_copy(uids.at[pl.ds(base, BPW)], uid_v)
    pltpu.sync_copy(iids.at[pl.ds(base, BPW)], iid_v)

    cbu = pltpu.async_copy(bu.at[uid_v], bu_v, sem_bu)
    cbi = pltpu.async_copy(bi.at[iid_v], bi_v, sem_bi)
    pltpu.sync_copy(gb, gb_v.at[pl.ds(0, 1)])

    def build(g, carry):
        o = g * LANES
        u16 = uid_v[pl.ds(o, LANES)]
        i16 = iid_v[pl.ds(o, LANES)]
        for f in range(EMBED_DIM):
            uidx[pl.ds(f * BPW + o, LANES)] = u16 + f * NROWS
            iidx[pl.ds(f * BPW + o, LANES)] = i16 + f * NROWS
        return carry

    lax.fori_loop(0, GROUPS, build, 0)

    cu = pltpu.async_copy(eut.at[uidx], ucols, sem_u)
    ci = pltpu.async_copy(eit.at[iidx], icols, sem_i)
    cu.wait()
    ci.wait()
    cbu.wait()
    cbi.wait()

    gbs = gb_v[...][0]

    def group(g, carry):
        o = g * LANES
        acc = jnp.zeros((LANES,), jnp.float32)
        for f in range(EMBED_DIM):
            acc = acc + (ucols[pl.ds(f * BPW + o, LANES)]
                         * icols[pl.ds(f * BPW + o, LANES)])
        out_v[pl.ds(o, LANES)] = (acc + bu_v[pl.ds(o, LANES)]
                                  + bi_v[pl.ds(o, LANES)] + gbs)
        return carry

    lax.fori_loop(0, GROUPS, group, 0)
    pltpu.sync_copy(out_v, out.at[pl.ds(base, BPW)])


@jax.jit
def _mf(user_ids, item_ids, embedding_users, embedding_items,
        bias_users, bias_items, global_bias):
    mesh = plsc.VectorSubcoreMesh(core_axis_name="c", subcore_axis_name="s")
    run = pl.kernel(
        _mf_body,
        mesh=mesh,
        out_type=jax.ShapeDtypeStruct((BATCH,), jnp.float32),
        compiler_params=pltpu.CompilerParams(
            needs_layout_passes=False, use_tc_tiling_on_sc=False),
        scratch_types=[
            pltpu.VMEM((BPW,), jnp.int32),
            pltpu.VMEM((BPW,), jnp.int32),
            pltpu.VMEM((BPW * EMBED_DIM,), jnp.int32),
            pltpu.VMEM((BPW * EMBED_DIM,), jnp.int32),
            pltpu.VMEM((BPW * EMBED_DIM,), jnp.float32),
            pltpu.VMEM((BPW * EMBED_DIM,), jnp.float32),
            pltpu.VMEM((BPW,), jnp.float32),
            pltpu.VMEM((BPW,), jnp.float32),
            pltpu.VMEM((LANES,), jnp.float32),
            pltpu.VMEM((BPW,), jnp.float32),
            pltpu.SemaphoreType.DMA,
            pltpu.SemaphoreType.DMA,
            pltpu.SemaphoreType.DMA,
            pltpu.SemaphoreType.DMA,
        ],
    )
    return run(user_ids, item_ids,
               embedding_users.T.reshape(-1), embedding_items.T.reshape(-1),
               bias_users, bias_items, global_bias)


def kernel(user_ids, item_ids, embedding_users, embedding_items,
           bias_users, bias_items, global_bias):
    return _mf(user_ids, item_ids, embedding_users, embedding_items,
               bias_users, bias_items, global_bias)


# speedup vs baseline: 5.7167x; 5.7167x over previous
import functools

import jax
import jax.numpy as jnp
from jax import lax
from jax.experimental import pallas as pl
from jax.experimental.pallas import tpu as pltpu
from jax.experimental.pallas import tpu_sc as plsc

BATCH = 16384
EMBED_DIM = 32
LANES = 16
NROWS = 1000000
QROWS = NROWS // 4
QDIM = 4 * EMBED_DIM

_info = plsc.get_sparse_core_info()
NC, NS = _info.num_cores, _info.num_subcores
NW = NC * NS
BPW = BATCH // NW
HALF = BPW // 2
HGROUPS = HALF // LANES


def _mf_body(uids, iids, eu4, ei4, bu, bi, gb, out,
             uid_v, iid_v, uq, iq, urows, irows, bu_v, bi_v, gb_v,
             out_v, sem_u, sem_i, sem_bu, sem_bi):
    wid = lax.axis_index("s") * NC + lax.axis_index("c")
    base = wid * BPW

    pltpu.sync_copy(uids.at[pl.ds(base, BPW)], uid_v)
    pltpu.sync_copy(iids.at[pl.ds(base, BPW)], iid_v)

    cbu = pltpu.async_copy(bu.at[uid_v], bu_v, sem_bu)
    cbi = pltpu.async_copy(bi.at[iid_v], bi_v, sem_bi)
    pltpu.sync_copy(gb, gb_v.at[pl.ds(0, 1)])

    lanes = lax.iota(jnp.int32, LANES)

    def half(h, carry):
        hb = h * HALF

        def build(g, c):
            o = g * LANES
            uq[pl.ds(o, LANES)] = jnp.right_shift(
                uid_v[pl.ds(hb + o, LANES)], 2)
            iq[pl.ds(o, LANES)] = jnp.right_shift(
                iid_v[pl.ds(hb + o, LANES)], 2)
            return c

        lax.fori_loop(0, HGROUPS, build, 0)

        cu = pltpu.async_copy(eu4.at[uq], urows, sem_u)
        ci = pltpu.async_copy(ei4.at[iq], irows, sem_i)
        cu.wait()
        ci.wait()

        def group(g, c):
            o = g * LANES
            row = o + lanes
            u16 = uid_v[pl.ds(hb + o, LANES)]
            i16 = iid_v[pl.ds(hb + o, LANES)]
            uoff = (u16 & 3) * EMBED_DIM
            ioff = (i16 & 3) * EMBED_DIM
            acc = jnp.zeros((LANES,), jnp.float32)
            for k in range(EMBED_DIM):
                f = (lanes + k) & (EMBED_DIM - 1)
                acc = acc + (plsc.load_gather(urows, [row, uoff + f])
                             * plsc.load_gather(irows, [row, ioff + f]))
            out_v[pl.ds(hb + o, LANES)] = acc
            return c

        lax.fori_loop(0, HGROUPS, group, 0)
        return carry

    lax.fori_loop(0, 2, half, 0)

    cbu.wait()
    cbi.wait()
    gbs = gb_v[...][0]

    def finish(g, carry):
        o = g * LANES
        out_v[pl.ds(o, LANES)] = (out_v[pl.ds(o, LANES)]
                                  + bu_v[pl.ds(o, LANES)]
                                  + bi_v[pl.ds(o, LANES)] + gbs)
        return carry

    lax.fori_loop(0, BPW // LANES, finish, 0)
    pltpu.sync_copy(out_v, out.at[pl.ds(base, BPW)])


@jax.jit
def _mf(user_ids, item_ids, embedding_users, embedding_items,
        bias_users, bias_items, global_bias):
    mesh = plsc.VectorSubcoreMesh(core_axis_name="c", subcore_axis_name="s")
    run = pl.kernel(
        _mf_body,
        mesh=mesh,
        out_type=jax.ShapeDtypeStruct((BATCH,), jnp.float32),
        compiler_params=pltpu.CompilerParams(
            needs_layout_passes=False, use_tc_tiling_on_sc=True),
        scratch_types=[
            pltpu.VMEM((BPW,), jnp.int32),
            pltpu.VMEM((BPW,), jnp.int32),
            pltpu.VMEM((HALF,), jnp.int32),
            pltpu.VMEM((HALF,), jnp.int32),
            pltpu.VMEM((HALF, QDIM), jnp.float32),
            pltpu.VMEM((HALF, QDIM), jnp.float32),
            pltpu.VMEM((BPW,), jnp.float32),
            pltpu.VMEM((BPW,), jnp.float32),
            pltpu.VMEM((LANES,), jnp.float32),
            pltpu.VMEM((BPW,), jnp.float32),
            pltpu.SemaphoreType.DMA,
            pltpu.SemaphoreType.DMA,
            pltpu.SemaphoreType.DMA,
            pltpu.SemaphoreType.DMA,
        ],
    )
    return run(user_ids, item_ids,
               embedding_users.reshape(QROWS, QDIM),
               embedding_items.reshape(QROWS, QDIM),
               bias_users, bias_items, global_bias)


def kernel(user_ids, item_ids, embedding_users, embedding_items,
           bias_users, bias_items, global_bias):
    return _mf(user_ids, item_ids, embedding_users, embedding_items,
               bias_users, bias_items, global_bias)


# speedup vs baseline: 22.5673x; 3.9476x over previous
import functools

import jax
import jax.numpy as jnp
from jax import lax
from jax.experimental import pallas as pl
from jax.experimental.pallas import tpu as pltpu
from jax.experimental.pallas import tpu_sc as plsc

BATCH = 16384
EMBED_DIM = 32
LANES = 16
NROWS = 1000000
NBLK = 128
FG = 8
NFG = EMBED_DIM // FG

_info = plsc.get_sparse_core_info()
NC, NS = _info.num_cores, _info.num_subcores
NW = NC * NS
BPW = BATCH // NW
CH = 4
NCHUNK = BPW // CH
TPC = CH * NFG


def _mf_body(uids, iids, eut, eit, bu, bi, gb, out,
             uid_v, iid_v, utiles, itiles,
             bu_v, bi_v, gb_v, outf_v, out_v,
             sem_u, sem_i, sem_bu, sem_bi):
    wid = lax.axis_index("s") * NC + lax.axis_index("c")
    base = wid * BPW

    pltpu.sync_copy(uids.at[pl.ds(base, BPW)], uid_v)
    pltpu.sync_copy(iids.at[pl.ds(base, BPW)], iid_v)

    cbu = pltpu.async_copy(bu.at[uid_v], bu_v, sem_bu)
    cbi = pltpu.async_copy(bi.at[iid_v], bi_v, sem_bi)
    pltpu.sync_copy(gb, gb_v.at[pl.ds(0, 1)])

    lanes = lax.iota(jnp.int32, LANES)
    quad = jnp.right_shift(lanes, 2)
    jlane = jnp.right_shift(lanes, 3)
    rlane = lanes & (FG - 1)

    def read_ids(c):
        uvec = plsc.load_gather(uid_v, [c * CH + quad])
        ivec = plsc.load_gather(iid_v, [c * CH + quad])
        return uvec, ivec

    def issue(c, buf):
        uvec, ivec = read_ids(c)
        for n in range(CH):
            uid = uvec[4 * n]
            iid = ivec[4 * n]
            uoff = pl.multiple_of(jnp.right_shift(uid, 7) * NBLK, NBLK)
            ioff = pl.multiple_of(jnp.right_shift(iid, 7) * NBLK, NBLK)
            for j in range(NFG):
                pltpu.async_copy(
                    eut.at[pl.ds(FG * j, FG), pl.ds(uoff, NBLK)],
                    utiles.at[buf, n * NFG + j], sem_u)
                pltpu.async_copy(
                    eit.at[pl.ds(FG * j, FG), pl.ds(ioff, NBLK)],
                    itiles.at[buf, n * NFG + j], sem_i)

    issue(0, 0)
    issue(1, 1)

    def chunk_body(g, acc):
        buf = g & 1
        pltpu.make_async_copy(
            eut.at[pl.ds(0, FG), pl.ds(0, TPC * NBLK)],
            utiles.at[buf], sem_u).wait()
        pltpu.make_async_copy(
            eit.at[pl.ds(0, FG), pl.ds(0, TPC * NBLK)],
            itiles.at[buf], sem_i).wait()

        bufv = jnp.full((LANES,), buf, jnp.int32)
        uvec, ivec = read_ids(g)
        for n in range(CH):
            ucol = jnp.full((LANES,), uvec[4 * n] & (NBLK - 1), jnp.int32)
            icol = jnp.full((LANES,), ivec[4 * n] & (NBLK - 1), jnp.int32)
            tb = n * NFG
            u_lo = plsc.load_gather(utiles, [bufv, tb + jlane, rlane, ucol])
            u_hi = plsc.load_gather(utiles, [bufv, tb + 2 + jlane, rlane, ucol])
            i_lo = plsc.load_gather(itiles, [bufv, tb + jlane, rlane, icol])
            i_hi = plsc.load_gather(itiles, [bufv, tb + 2 + jlane, rlane, icol])
            s = jnp.sum(u_lo * i_lo + u_hi * i_hi)
            acc = jnp.where(lanes == (g & 3) * CH + n, s, acc)

        @pl.when((g & 3) == 3)
        def _():
            outf_v[pl.ds(jnp.right_shift(g, 2) * LANES, LANES)] = acc

        @pl.when(g < NCHUNK - 2)
        def _():
            issue(g + 2, buf)

        return jnp.where((g & 3) == 3, jnp.zeros((LANES,), jnp.float32), acc)

    lax.fori_loop(0, NCHUNK, chunk_body, jnp.zeros((LANES,), jnp.float32))

    cbu.wait()
    cbi.wait()
    gbs = gb_v[...][0]

    def finish(k, carry):
        o = k * LANES
        out_v[pl.ds(o, LANES)] = (outf_v[pl.ds(o, LANES)]
                                  + bu_v[pl.ds(o, LANES)]
                                  + bi_v[pl.ds(o, LANES)] + gbs)
        return carry

    lax.fori_loop(0, BPW // LANES, finish, 0)
    pltpu.sync_copy(out_v, out.at[pl.ds(base, BPW)])


@jax.jit
def _mf(user_ids, item_ids, embedding_users, embedding_items,
        bias_users, bias_items, global_bias):
    mesh = plsc.VectorSubcoreMesh(core_axis_name="c", subcore_axis_name="s")
    run = pl.kernel(
        _mf_body,
        mesh=mesh,
        out_type=jax.ShapeDtypeStruct((BATCH,), jnp.float32),
        compiler_params=pltpu.CompilerParams(
            needs_layout_passes=False, use_tc_tiling_on_sc=True),
        scratch_types=[
            pltpu.VMEM((BPW,), jnp.int32),
            pltpu.VMEM((BPW,), jnp.int32),
            pltpu.VMEM((2, TPC, FG, NBLK), jnp.float32),
            pltpu.VMEM((2, TPC, FG, NBLK), jnp.float32),
            pltpu.VMEM((BPW,), jnp.float32),
            pltpu.VMEM((BPW,), jnp.float32),
            pltpu.VMEM((LANES,), jnp.float32),
            pltpu.VMEM((BPW,), jnp.float32),
            pltpu.VMEM((BPW,), jnp.float32),
            pltpu.SemaphoreType.DMA,
            pltpu.SemaphoreType.DMA,
            pltpu.SemaphoreType.DMA,
            pltpu.SemaphoreType.DMA,
        ],
    )
    return run(user_ids, item_ids, embedding_users.T, embedding_items.T,
               bias_users, bias_items, global_bias)


def kernel(user_ids, item_ids, embedding_users, embedding_items,
           bias_users, bias_items, global_bias):
    return _mf(user_ids, item_ids, embedding_users, embedding_items,
               bias_users, bias_items, global_bias)


# speedup vs baseline: 22.6465x; 1.0035x over previous
import functools

import jax
import jax.numpy as jnp
from jax import lax
from jax.experimental import pallas as pl
from jax.experimental.pallas import tpu as pltpu
from jax.experimental.pallas import tpu_sc as plsc

BATCH = 16384
EMBED_DIM = 32
LANES = 16
NROWS = 1000000
NBLK = 128
FG = 8
NFG = EMBED_DIM // FG

_info = plsc.get_sparse_core_info()
NC, NS = _info.num_cores, _info.num_subcores
NW = NC * NS
BPW = BATCH // NW
CH = 4
NCHUNK = BPW // CH
TPC = CH * NFG


def _mf_body(uids, iids, eut, eit, bu, bi, gb, out,
             uid_v, iid_v, utiles, itiles,
             bu_v, bi_v, gb_v, outf_v, out_v,
             sem_u, sem_i, sem_bu, sem_bi):
    wid = lax.axis_index("s") * NC + lax.axis_index("c")
    base = wid * BPW

    pltpu.sync_copy(uids.at[pl.ds(base, BPW)], uid_v)
    pltpu.sync_copy(iids.at[pl.ds(base, BPW)], iid_v)

    cbu = pltpu.async_copy(bu.at[uid_v], bu_v, sem_bu)
    cbi = pltpu.async_copy(bi.at[iid_v], bi_v, sem_bi)
    pltpu.sync_copy(gb, gb_v.at[pl.ds(0, 1)])

    lanes = lax.iota(jnp.int32, LANES)
    quad = jnp.right_shift(lanes, 2)

    def read_ids(c):
        uvec = plsc.load_gather(uid_v, [c * CH + quad])
        ivec = plsc.load_gather(iid_v, [c * CH + quad])
        return uvec, ivec

    def issue(c, buf):
        uvec, ivec = read_ids(c)
        for n in range(CH):
            uid = uvec[4 * n]
            iid = ivec[4 * n]
            uoff = pl.multiple_of(jnp.right_shift(uid, 7) * NBLK, NBLK)
            ioff = pl.multiple_of(jnp.right_shift(iid, 7) * NBLK, NBLK)
            pltpu.async_copy(
                eut.at[pl.ds(0, EMBED_DIM), pl.ds(uoff, NBLK)],
                utiles.at[buf, n], sem_u)
            pltpu.async_copy(
                eit.at[pl.ds(0, EMBED_DIM), pl.ds(ioff, NBLK)],
                itiles.at[buf, n], sem_i)

    issue(0, 0)
    issue(1, 1)

    def chunk_body(g, acc):
        buf = g & 1
        pltpu.make_async_copy(
            eut.at[pl.ds(0, EMBED_DIM), pl.ds(0, CH * NBLK)],
            utiles.at[buf], sem_u).wait()
        pltpu.make_async_copy(
            eit.at[pl.ds(0, EMBED_DIM), pl.ds(0, CH * NBLK)],
            itiles.at[buf], sem_i).wait()

        bufv = jnp.full((LANES,), buf, jnp.int32)
        uvec, ivec = read_ids(g)
        for n in range(CH):
            nv = jnp.full((LANES,), n, jnp.int32)
            ucol = jnp.full((LANES,), uvec[4 * n] & (NBLK - 1), jnp.int32)
            icol = jnp.full((LANES,), ivec[4 * n] & (NBLK - 1), jnp.int32)
            u_lo = plsc.load_gather(utiles, [bufv, nv, lanes, ucol])
            u_hi = plsc.load_gather(utiles, [bufv, nv, LANES + lanes, ucol])
            i_lo = plsc.load_gather(itiles, [bufv, nv, lanes, icol])
            i_hi = plsc.load_gather(itiles, [bufv, nv, LANES + lanes, icol])
            s = jnp.sum(u_lo * i_lo + u_hi * i_hi)
            acc = jnp.where(lanes == (g & 3) * CH + n, s, acc)

        @pl.when((g & 3) == 3)
        def _():
            outf_v[pl.ds(jnp.right_shift(g, 2) * LANES, LANES)] = acc

        @pl.when(g < NCHUNK - 2)
        def _():
            issue(g + 2, buf)

        return jnp.where((g & 3) == 3, jnp.zeros((LANES,), jnp.float32), acc)

    lax.fori_loop(0, NCHUNK, chunk_body, jnp.zeros((LANES,), jnp.float32))

    cbu.wait()
    cbi.wait()
    gbs = gb_v[...][0]

    def finish(k, carry):
        o = k * LANES
        out_v[pl.ds(o, LANES)] = (outf_v[pl.ds(o, LANES)]
                                  + bu_v[pl.ds(o, LANES)]
                                  + bi_v[pl.ds(o, LANES)] + gbs)
        return carry

    lax.fori_loop(0, BPW // LANES, finish, 0)
    pltpu.sync_copy(out_v, out.at[pl.ds(base, BPW)])


@jax.jit
def _mf(user_ids, item_ids, embedding_users, embedding_items,
        bias_users, bias_items, global_bias):
    mesh = plsc.VectorSubcoreMesh(core_axis_name="c", subcore_axis_name="s")
    run = pl.kernel(
        _mf_body,
        mesh=mesh,
        out_type=jax.ShapeDtypeStruct((BATCH,), jnp.float32),
        compiler_params=pltpu.CompilerParams(
            needs_layout_passes=False, use_tc_tiling_on_sc=True),
        scratch_types=[
            pltpu.VMEM((BPW,), jnp.int32),
            pltpu.VMEM((BPW,), jnp.int32),
            pltpu.VMEM((2, CH, EMBED_DIM, NBLK), jnp.float32),
            pltpu.VMEM((2, CH, EMBED_DIM, NBLK), jnp.float32),
            pltpu.VMEM((BPW,), jnp.float32),
            pltpu.VMEM((BPW,), jnp.float32),
            pltpu.VMEM((LANES,), jnp.float32),
            pltpu.VMEM((BPW,), jnp.float32),
            pltpu.VMEM((BPW,), jnp.float32),
            pltpu.SemaphoreType.DMA,
            pltpu.SemaphoreType.DMA,
            pltpu.SemaphoreType.DMA,
            pltpu.SemaphoreType.DMA,
        ],
    )
    return run(user_ids, item_ids, embedding_users.T, embedding_items.T,
               bias_users, bias_items, global_bias)


def kernel(user_ids, item_ids, embedding_users, embedding_items,
           bias_users, bias_items, global_bias):
    return _mf(user_ids, item_ids, embedding_users, embedding_items,
               bias_users, bias_items, global_bias)


# speedup vs baseline: 24.7575x; 1.0932x over previous
import functools

import jax
import jax.numpy as jnp
from jax import lax
from jax.experimental import pallas as pl
from jax.experimental.pallas import tpu as pltpu
from jax.experimental.pallas import tpu_sc as plsc

BATCH = 16384
EMBED_DIM = 32
LANES = 16
NROWS = 1000000
NBLK = 128
FG = 8
NFG = EMBED_DIM // FG

_info = plsc.get_sparse_core_info()
NC, NS = _info.num_cores, _info.num_subcores
NW = NC * NS
BPW = BATCH // NW
CH = 4
NCHUNK = BPW // CH
TPC = CH * NFG


def _mf_body(uids, iids, eut, eit, bu, bi, gb, out,
             uid_v, iid_v, utiles, itiles,
             bu_v, bi_v, gb_v, outf_v, out_v,
             sem_u, sem_i, sem_bu, sem_bi):
    wid = lax.axis_index("s") * NC + lax.axis_index("c")
    base = wid * BPW

    pltpu.sync_copy(uids.at[pl.ds(base, BPW)], uid_v)
    pltpu.sync_copy(iids.at[pl.ds(base, BPW)], iid_v)

    cbu = pltpu.async_copy(bu.at[uid_v], bu_v, sem_bu)
    cbi = pltpu.async_copy(bi.at[iid_v], bi_v, sem_bi)
    pltpu.sync_copy(gb, gb_v.at[pl.ds(0, 1)])

    lanes = lax.iota(jnp.int32, LANES)
    quad = jnp.right_shift(lanes, 2)

    def read_ids(c):
        uvec = plsc.load_gather(uid_v, [c * CH + quad])
        ivec = plsc.load_gather(iid_v, [c * CH + quad])
        return uvec, ivec

    def issue(c, buf):
        uvec, ivec = read_ids(c)
        for n in range(CH):
            uid = uvec[4 * n]
            iid = ivec[4 * n]
            uoff = pl.multiple_of(jnp.right_shift(uid, 7) * NBLK, NBLK)
            ioff = pl.multiple_of(jnp.right_shift(iid, 7) * NBLK, NBLK)
            pltpu.async_copy(
                eut.at[pl.ds(0, EMBED_DIM), pl.ds(uoff, NBLK)],
                utiles.at[buf, n], sem_u)
            pltpu.async_copy(
                eit.at[pl.ds(0, EMBED_DIM), pl.ds(ioff, NBLK)],
                itiles.at[buf, n], sem_i)

    issue(0, 0)
    issue(1, 1)
    issue(2, 2)

    def chunk_body(g, acc):
        buf = lax.rem(g, 3)
        pltpu.make_async_copy(
            eut.at[pl.ds(0, EMBED_DIM), pl.ds(0, CH * NBLK)],
            utiles.at[buf], sem_u).wait()
        pltpu.make_async_copy(
            eit.at[pl.ds(0, EMBED_DIM), pl.ds(0, CH * NBLK)],
            itiles.at[buf], sem_i).wait()

        bufv = jnp.full((LANES,), buf, jnp.int32)
        uvec, ivec = read_ids(g)
        for n in range(CH):
            nv = jnp.full((LANES,), n, jnp.int32)
            ucol = jnp.full((LANES,), uvec[4 * n] & (NBLK - 1), jnp.int32)
            icol = jnp.full((LANES,), ivec[4 * n] & (NBLK - 1), jnp.int32)
            u_lo = plsc.load_gather(utiles, [bufv, nv, lanes, ucol])
            u_hi = plsc.load_gather(utiles, [bufv, nv, LANES + lanes, ucol])
            i_lo = plsc.load_gather(itiles, [bufv, nv, lanes, icol])
            i_hi = plsc.load_gather(itiles, [bufv, nv, LANES + lanes, icol])
            s = jnp.sum(u_lo * i_lo + u_hi * i_hi)
            acc = jnp.where(lanes == (g & 3) * CH + n, s, acc)

        @pl.when((g & 3) == 3)
        def _():
            outf_v[pl.ds(jnp.right_shift(g, 2) * LANES, LANES)] = acc

        @pl.when(g < NCHUNK - 3)
        def _():
            issue(g + 3, buf)

        return jnp.where((g & 3) == 3, jnp.zeros((LANES,), jnp.float32), acc)

    lax.fori_loop(0, NCHUNK, chunk_body, jnp.zeros((LANES,), jnp.float32))

    cbu.wait()
    cbi.wait()
    gbs = gb_v[...][0]

    def finish(k, carry):
        o = k * LANES
        out_v[pl.ds(o, LANES)] = (outf_v[pl.ds(o, LANES)]
                                  + bu_v[pl.ds(o, LANES)]
                                  + bi_v[pl.ds(o, LANES)] + gbs)
        return carry

    lax.fori_loop(0, BPW // LANES, finish, 0)
    pltpu.sync_copy(out_v, out.at[pl.ds(base, BPW)])


@jax.jit
def _mf(user_ids, item_ids, embedding_users, embedding_items,
        bias_users, bias_items, global_bias):
    mesh = plsc.VectorSubcoreMesh(core_axis_name="c", subcore_axis_name="s")
    run = pl.kernel(
        _mf_body,
        mesh=mesh,
        out_type=jax.ShapeDtypeStruct((BATCH,), jnp.float32),
        compiler_params=pltpu.CompilerParams(
            needs_layout_passes=False, use_tc_tiling_on_sc=True),
        scratch_types=[
            pltpu.VMEM((BPW,), jnp.int32),
            pltpu.VMEM((BPW,), jnp.int32),
            pltpu.VMEM((3, CH, EMBED_DIM, NBLK), jnp.float32),
            pltpu.VMEM((3, CH, EMBED_DIM, NBLK), jnp.float32),
            pltpu.VMEM((BPW,), jnp.float32),
            pltpu.VMEM((BPW,), jnp.float32),
            pltpu.VMEM((LANES,), jnp.float32),
            pltpu.VMEM((BPW,), jnp.float32),
            pltpu.VMEM((BPW,), jnp.float32),
            pltpu.SemaphoreType.DMA,
            pltpu.SemaphoreType.DMA,
            pltpu.SemaphoreType.DMA,
            pltpu.SemaphoreType.DMA,
        ],
    )
    return run(user_ids, item_ids, embedding_users.T, embedding_items.T,
               bias_users, bias_items, global_bias)


def kernel(user_ids, item_ids, embedding_users, embedding_items,
           bias_users, bias_items, global_bias):
    return _mf(user_ids, item_ids, embedding_users, embedding_items,
               bias_users, bias_items, global_bias)
